# baseline (speedup 1.0000x reference)
import functools
import math

import jax
import jax.numpy as jnp
from jax import lax
from jax.experimental import pallas as pl
from jax.experimental.pallas import tpu as pltpu
from jax.experimental.pallas import tpu_sc as plsc

_N = 10000
_D = 128
_E = 320000
_ALPHA = 0.1
_BETA = float(math.log(0.5 / 2.0 + 1.0))

_NC = 2
_NS = 16
_NW = _NC * _NS
_L = 16

_S = 96
_NCH = 108
_NCHT = _NCH + 2
_EPT = _NCH * _S
_E2 = _NW * _EPT
_NB = 4

_RPT = 624
_TAIL = _N - _NS * _RPT


def _splat(vec, i):
    idx = jnp.full((_L,), i, jnp.int32)
    dnums = lax.GatherDimensionNumbers(
        offset_dims=(), collapsed_slice_dims=(0,), start_index_map=(0,))
    return lax.gather(vec, idx[:, None], dnums, (1,),
                      mode=lax.GatherScatterMode.PROMISE_IN_BOUNDS)


def _sc_gather_scatter(x, pk, wf, zeros):
    mesh = plsc.VectorSubcoreMesh(
        core_axis_name="c", subcore_axis_name="s",
        num_cores=_NC, num_subcores=_NS)

    @functools.partial(
        pl.kernel,
        out_type=jax.ShapeDtypeStruct((_NC, _N, _D), jnp.float32),
        mesh=mesh,
        scratch_types=[
            pltpu.VMEM((_NB, 2, _S), jnp.int32),
            pltpu.VMEM((_NB, _S), jnp.float32),
            pltpu.VMEM((_S, _D), jnp.float32),
            pltpu.VMEM((_S, _D), jnp.float32),
            pltpu.VMEM((_S, _D), jnp.float32),
            pltpu.VMEM((_S, _D), jnp.float32),
            pltpu.VMEM_SHARED((_N, _D), jnp.float32),
            pltpu.SemaphoreType.DMA,
            (pltpu.SemaphoreType.DMA,) * _NB,
            (pltpu.SemaphoreType.DMA,) * _NB,
            (pltpu.SemaphoreType.DMA,) * _NB,
        ],
    )
    def k(x_hbm, pk_hbm, wf_hbm, z_hbm, out_hbm,
          pk_v, w_v, rows0, rows1, rows2, rows3, agg_sh,
          zsem, isems, gsems, ssems):
        c = lax.axis_index("c")
        s = lax.axis_index("s")
        gwid = s * _NC + c
        cbase = gwid * _NCHT

        rows = (rows0, rows1, rows2, rows3)

        zc = pltpu.async_copy(z_hbm.at[pl.ds(s * _RPT, _RPT)],
                              agg_sh.at[pl.ds(s * _RPT, _RPT)], zsem)

        def fire_idx(kk, b):
            pltpu.async_copy(pk_hbm.at[cbase + kk], pk_v.at[b], isems[b])
            pltpu.async_copy(wf_hbm.at[pl.ds((cbase + kk) * _S, _S)],
                             w_v.at[b], isems[b])

        def wait_idx(b):
            pltpu.make_async_copy(pk_hbm.at[0], pk_v.at[b], isems[b]).wait()
            pltpu.make_async_copy(wf_hbm.at[pl.ds(0, _S)], w_v.at[b],
                                  isems[b]).wait()

        def fire_gather(b):
            pltpu.async_copy(x_hbm.at[pl.ds(b * 96, _S)], rows[b], gsems[b])

        def wait_gather(b):
            pltpu.make_async_copy(x_hbm.at[pl.ds(0, _S)], rows[b],
                                  gsems[b]).wait()

        def fire_scatter(b):
            pass

        def wait_scatter(b):
            pass

        def scale(b, kk):
            def grp(g, carry):
                w16 = w_v[b, pl.ds(g * _L, _L)]
                for e in range(_L):
                    ws = _splat(w16, e)
                    row = g * _L + e
                    for j in range(_D // _L):
                        rows[b][row, pl.ds(j * _L, _L)] = (
                            rows[b][row, pl.ds(j * _L, _L)] * ws)
                return carry
            lax.fori_loop(0, _S // _L, grp, 0)

        fire_idx(0, 0)
        fire_idx(1, 1)
        wait_idx(0)
        fire_gather(0)

        zc.wait()

        @pl.when(s == _NS - 1)
        def _zero_tail():
            pltpu.async_copy(z_hbm.at[pl.ds(_NS * _RPT, _TAIL)],
                             agg_sh.at[pl.ds(_NS * _RPT, _TAIL)], zsem).wait()

        plsc.subcore_barrier()

        def body(i, carry):
            for h in range(_NB):
                kk = _NB * i + h
                b = h
                bn = (h + 1) % _NB
                b2 = (h + 2) % _NB

                if h < 2:
                    @pl.when(kk >= 2)
                    def _ws():
                        wait_scatter(b2)
                else:
                    wait_scatter(b2)
                wait_idx(bn)
                fire_gather(bn)
                wait_gather(b)
                scale(b, kk)
                fire_scatter(b)
                fire_idx(kk + 2, b2)
            return carry

        lax.fori_loop(0, _NCH // _NB, body, 0)

        wait_scatter((_NCH - 2) % _NB)
        wait_scatter((_NCH - 1) % _NB)
        wait_gather(_NCH % _NB)
        wait_idx((_NCH + 1) % _NB)

        plsc.subcore_barrier()
        pltpu.sync_copy(agg_sh.at[pl.ds(s * _RPT, _RPT)],
                        out_hbm.at[c, pl.ds(s * _RPT, _RPT)])

        @pl.when(s == _NS - 1)
        def _out_tail():
            pltpu.sync_copy(agg_sh.at[pl.ds(_NS * _RPT, _TAIL)],
                            out_hbm.at[c, pl.ds(_NS * _RPT, _TAIL)])

    return k(x, pk, wf, zeros)


def _tc_epilogue(p0, p1, x0, xin, w1):
    blk = 1000

    def body(p0_ref, p1_ref, x0_ref, xin_ref, w1_ref, o_ref):
        agg = p0_ref[...] + p1_ref[...]
        h = (1.0 - _ALPHA) * agg + _ALPHA * x0_ref[...]
        hw = jnp.dot(h, w1_ref[...], preferred_element_type=jnp.float32)
        o_ref[...] = jnp.maximum(
            (1.0 - _BETA) * h + _BETA * hw + xin_ref[...], 0.0)

    return pl.pallas_call(
        body,
        grid=(_N // blk,),
        in_specs=[
            pl.BlockSpec((blk, _D), lambda i: (i, 0)),
            pl.BlockSpec((blk, _D), lambda i: (i, 0)),
            pl.BlockSpec((blk, _D), lambda i: (i, 0)),
            pl.BlockSpec((blk, _D), lambda i: (i, 0)),
            pl.BlockSpec((_D, _D), lambda i: (0, 0)),
        ],
        out_specs=pl.BlockSpec((blk, _D), lambda i: (i, 0)),
        out_shape=jax.ShapeDtypeStruct((_N, _D), jnp.float32),
    )(p0, p1, x0, xin, w1)


def kernel(x, x_0, edge_index, edge_weight, W1):
    src = edge_index[0].astype(jnp.int32)
    dst = edge_index[1].astype(jnp.int32)

    pad = _E2 - _E
    fill = (jnp.arange(pad, dtype=jnp.int32) * 13) % _N
    src_p = jnp.concatenate([src, fill]).reshape(_NW, _NCH, _S)
    dst_p = jnp.concatenate([dst, fill]).reshape(_NW, _NCH, _S)
    pk = jnp.stack([src_p, dst_p], axis=2)

    dummy_i = jnp.broadcast_to(
        ((jnp.arange(2 * _S, dtype=jnp.int32) * 37) % _N).reshape(2, _S)[
            None, :, None, :],
        (_NW, 2, 1, _S))
    dummy = jnp.concatenate([dummy_i, dummy_i], axis=2)
    pk = jnp.concatenate([pk, dummy], axis=1).reshape(_NW * _NCHT, 2, _S)

    w_p = jnp.concatenate(
        [edge_weight, jnp.zeros((pad,), jnp.float32)]).reshape(_NW, _NCH, _S)
    wf = jnp.concatenate(
        [w_p, jnp.zeros((_NW, 2, _S), jnp.float32)], axis=1).reshape(-1)

    zeros = jnp.zeros((_N, _D), jnp.float32)
    partials = _sc_gather_scatter(x, pk, wf, zeros)
    return _tc_epilogue(partials[0], partials[1], x_0, x, W1)


# speedup vs baseline: 1.0921x; 1.0921x over previous
import functools
import math

import jax
import jax.numpy as jnp
from jax import lax
from jax.experimental import pallas as pl
from jax.experimental.pallas import tpu as pltpu
from jax.experimental.pallas import tpu_sc as plsc

_N = 10000
_D = 128
_E = 320000
_ALPHA = 0.1
_BETA = float(math.log(0.5 / 2.0 + 1.0))

_NC = 2
_NS = 16
_NW = _NC * _NS
_L = 16

_S = 96
_NCH = 108
_NCHT = _NCH + 3
_EPT = _NCH * _S
_E2 = _NW * _EPT

_RPT = 624
_TAIL = _N - _NS * _RPT


def _splat(vec, i):
    idx = jnp.full((_L,), i, jnp.int32)
    dnums = lax.GatherDimensionNumbers(
        offset_dims=(), collapsed_slice_dims=(0,), start_index_map=(0,))
    return lax.gather(vec, idx[:, None], dnums, (1,),
                      mode=lax.GatherScatterMode.PROMISE_IN_BOUNDS)


def _sc_gather_scatter(x, pk, wf, zeros):
    mesh = plsc.VectorSubcoreMesh(
        core_axis_name="c", subcore_axis_name="s",
        num_cores=_NC, num_subcores=_NS)

    @functools.partial(
        pl.kernel,
        out_type=jax.ShapeDtypeStruct((_NC, _N, _D), jnp.float32),
        mesh=mesh,
        scratch_types=[
            pltpu.VMEM((6, 2, _S), jnp.int32),
            pltpu.VMEM((6, _S), jnp.float32),
            pltpu.VMEM((_S, _D), jnp.float32),
            pltpu.VMEM((_S, _D), jnp.float32),
            pltpu.VMEM((_S, _D), jnp.float32),
            pltpu.VMEM_SHARED((_N, _D), jnp.float32),
            pltpu.SemaphoreType.DMA,
            (pltpu.SemaphoreType.DMA,) * 6,
            (pltpu.SemaphoreType.DMA,) * 3,
            (pltpu.SemaphoreType.DMA,) * 3,
        ],
    )
    def k(x_hbm, pk_hbm, wf_hbm, z_hbm, out_hbm,
          pk_v, w_v, r0, r1, r2, agg_sh,
          zsem, isems, gsems, ssems):
        c = lax.axis_index("c")
        s = lax.axis_index("s")
        gwid = s * _NC + c
        cbase = gwid * _NCHT

        rows = (r0, r1, r2)

        zc = pltpu.async_copy(z_hbm.at[pl.ds(s * _RPT, _RPT)],
                              agg_sh.at[pl.ds(s * _RPT, _RPT)], zsem)

        def fire_idx(kk, b6):
            pltpu.async_copy(pk_hbm.at[cbase + kk], pk_v.at[b6], isems[b6])
            pltpu.async_copy(wf_hbm.at[pl.ds((cbase + kk) * _S, _S)],
                             w_v.at[b6], isems[b6])

        def wait_idx(b6):
            pltpu.make_async_copy(pk_hbm.at[0], pk_v.at[b6], isems[b6]).wait()
            pltpu.make_async_copy(wf_hbm.at[pl.ds(0, _S)], w_v.at[b6],
                                  isems[b6]).wait()

        def fire_gather(i6, b4):
            pltpu.async_copy(x_hbm.at[pk_v.at[i6, 0]], rows[b4], gsems[b4])

        def wait_gather(b4):
            pltpu.make_async_copy(x_hbm.at[pl.ds(0, _S)], rows[b4],
                                  gsems[b4]).wait()

        def fire_scatter(b4, b6):
            pltpu.async_copy(rows[b4], agg_sh.at[pk_v.at[b6, 1]], ssems[b4],
                             add=True)

        def wait_scatter(b4):
            pltpu.make_async_copy(rows[b4], agg_sh.at[pl.ds(0, _S)],
                                  ssems[b4]).wait()

        def scale(b4, w6):
            def grp(g, carry):
                w16 = w_v[w6, pl.ds(g * _L, _L)]
                for e in range(_L):
                    ws = _splat(w16, e)
                    row = g * _L + e
                    for j in range(_D // _L):
                        rows[b4][row, pl.ds(j * _L, _L)] = (
                            rows[b4][row, pl.ds(j * _L, _L)] * ws)
                return carry
            lax.fori_loop(0, _S // _L, grp, 0)

        fire_idx(0, 0)
        fire_idx(1, 1)
        fire_idx(2, 2)
        wait_idx(0)
        fire_gather(0, 0)
        wait_idx(1)
        fire_gather(1, 1)

        zc.wait()

        @pl.when(s == _NS - 1)
        def _zero_tail():
            pltpu.async_copy(z_hbm.at[pl.ds(_NS * _RPT, _TAIL)],
                             agg_sh.at[pl.ds(_NS * _RPT, _TAIL)], zsem).wait()

        plsc.subcore_barrier()

        def body(i, carry):
            for h in range(6):
                kk = 6 * i + h
                b3 = h % 3
                g3 = (h + 2) % 3
                b6 = h % 6
                n6 = (h + 2) % 6

                if h < 1:
                    @pl.when(kk >= 1)
                    def _ws():
                        wait_scatter(g3)
                else:
                    wait_scatter(g3)
                wait_idx(n6)
                fire_gather(n6, g3)
                wait_gather(b3)
                scale(b3, b6)
                fire_scatter(b3, b6)
                fire_idx(kk + 3, (h + 3) % 6)
            return carry

        lax.fori_loop(0, _NCH // 6, body, 0)

        wait_scatter((_NCH - 1) % 3)
        wait_gather(_NCH % 3)
        wait_gather((_NCH + 1) % 3)
        wait_idx((_NCH + 2) % 6)

        plsc.subcore_barrier()
        pltpu.sync_copy(agg_sh.at[pl.ds(s * _RPT, _RPT)],
                        out_hbm.at[c, pl.ds(s * _RPT, _RPT)])

        @pl.when(s == _NS - 1)
        def _out_tail():
            pltpu.sync_copy(agg_sh.at[pl.ds(_NS * _RPT, _TAIL)],
                            out_hbm.at[c, pl.ds(_NS * _RPT, _TAIL)])

    return k(x, pk, wf, zeros)


def _tc_epilogue(p0, p1, x0, xin, w1):
    blk = 1000

    def body(p0_ref, p1_ref, x0_ref, xin_ref, w1_ref, o_ref):
        agg = p0_ref[...] + p1_ref[...]
        h = (1.0 - _ALPHA) * agg + _ALPHA * x0_ref[...]
        hw = jnp.dot(h, w1_ref[...], preferred_element_type=jnp.float32)
        o_ref[...] = jnp.maximum(
            (1.0 - _BETA) * h + _BETA * hw + xin_ref[...], 0.0)

    return pl.pallas_call(
        body,
        grid=(_N // blk,),
        in_specs=[
            pl.BlockSpec((blk, _D), lambda i: (i, 0)),
            pl.BlockSpec((blk, _D), lambda i: (i, 0)),
            pl.BlockSpec((blk, _D), lambda i: (i, 0)),
            pl.BlockSpec((blk, _D), lambda i: (i, 0)),
            pl.BlockSpec((_D, _D), lambda i: (0, 0)),
        ],
        out_specs=pl.BlockSpec((blk, _D), lambda i: (i, 0)),
        out_shape=jax.ShapeDtypeStruct((_N, _D), jnp.float32),
    )(p0, p1, x0, xin, w1)


def kernel(x, x_0, edge_index, edge_weight, W1):
    src = edge_index[0].astype(jnp.int32)
    dst = edge_index[1].astype(jnp.int32)

    pad = _E2 - _E
    fill = (jnp.arange(pad, dtype=jnp.int32) * 13) % _N
    src_p = jnp.concatenate([src, fill]).reshape(_NW, _NCH, _S)
    dst_p = jnp.concatenate([dst, fill]).reshape(_NW, _NCH, _S)
    pk = jnp.stack([src_p, dst_p], axis=2)

    dummy_i = jnp.broadcast_to(
        ((jnp.arange(3 * _S, dtype=jnp.int32) * 37) % _N).reshape(3, _S)[
            None, :, None, :],
        (_NW, 3, 1, _S))
    dummy = jnp.concatenate([dummy_i, dummy_i], axis=2)
    pk = jnp.concatenate([pk, dummy], axis=1).reshape(_NW * _NCHT, 2, _S)

    w_p = jnp.concatenate(
        [edge_weight, jnp.zeros((pad,), jnp.float32)]).reshape(_NW, _NCH, _S)
    wf = jnp.concatenate(
        [w_p, jnp.zeros((_NW, 3, _S), jnp.float32)], axis=1).reshape(-1)

    zeros = jnp.zeros((_N, _D), jnp.float32)
    partials = _sc_gather_scatter(x, pk, wf, zeros)
    return _tc_epilogue(partials[0], partials[1], x_0, x, W1)


# speedup vs baseline: 1.1621x; 1.0641x over previous
import functools
import math

import jax
import jax.numpy as jnp
from jax import lax
from jax.experimental import pallas as pl
from jax.experimental.pallas import tpu as pltpu
from jax.experimental.pallas import tpu_sc as plsc

_N = 10000
_D = 128
_E = 320000
_ALPHA = 0.1
_BETA = float(math.log(0.5 / 2.0 + 1.0))

_NC = 2
_NS = 16
_NW = _NC * _NS
_L = 16

_S = 96
_NCH = 108
_NCHT = _NCH + 3
_EPT = _NCH * _S
_E2 = _NW * _EPT

_RPT = 624
_TAIL = _N - _NS * _RPT


def _splat(vec, i):
    idx = jnp.full((_L,), i, jnp.int32)
    dnums = lax.GatherDimensionNumbers(
        offset_dims=(), collapsed_slice_dims=(0,), start_index_map=(0,))
    return lax.gather(vec, idx[:, None], dnums, (1,),
                      mode=lax.GatherScatterMode.PROMISE_IN_BOUNDS)


def _sc_gather_scatter(x, srcf, dstf, wf, zeros):
    mesh = plsc.VectorSubcoreMesh(
        core_axis_name="c", subcore_axis_name="s",
        num_cores=_NC, num_subcores=_NS)

    @functools.partial(
        pl.kernel,
        out_type=jax.ShapeDtypeStruct((_NC, _N, _D), jnp.float32),
        mesh=mesh,
        scratch_types=[
            pltpu.VMEM((6, 2, _S), jnp.int32),
            pltpu.VMEM((6, _S), jnp.float32),
            pltpu.VMEM((_S, _D), jnp.float32),
            pltpu.VMEM((_S, _D), jnp.float32),
            pltpu.VMEM((_S, _D), jnp.float32),
            pltpu.VMEM_SHARED((_N, _D), jnp.float32),
            pltpu.SemaphoreType.DMA,
            (pltpu.SemaphoreType.DMA,) * 6,
            (pltpu.SemaphoreType.DMA,) * 3,
            (pltpu.SemaphoreType.DMA,) * 3,
        ],
    )
    def k(x_hbm, src_hbm, dst_hbm, wf_hbm, z_hbm, out_hbm,
          pk_v, w_v, r0, r1, r2, agg_sh,
          zsem, isems, gsems, ssems):
        c = lax.axis_index("c")
        s = lax.axis_index("s")
        gwid = s * _NC + c
        ebase = gwid * _EPT

        rows = (r0, r1, r2)

        zc = pltpu.async_copy(z_hbm.at[pl.ds(s * _RPT, _RPT)],
                              agg_sh.at[pl.ds(s * _RPT, _RPT)], zsem)

        def fire_idx(kk, b6):
            off = jnp.where(kk < _NCH, ebase + kk * _S,
                            _E2 + (kk - _NCH) * _S)
            pltpu.async_copy(src_hbm.at[pl.ds(off, _S)], pk_v.at[b6, 0],
                             isems[b6])
            pltpu.async_copy(dst_hbm.at[pl.ds(off, _S)], pk_v.at[b6, 1],
                             isems[b6])
            pltpu.async_copy(wf_hbm.at[pl.ds(off, _S)], w_v.at[b6], isems[b6])

        def wait_idx(b6):
            pltpu.make_async_copy(src_hbm.at[pl.ds(0, _S)], pk_v.at[b6, 0],
                                  isems[b6]).wait()
            pltpu.make_async_copy(src_hbm.at[pl.ds(0, _S)], pk_v.at[b6, 1],
                                  isems[b6]).wait()
            pltpu.make_async_copy(wf_hbm.at[pl.ds(0, _S)], w_v.at[b6],
                                  isems[b6]).wait()

        def fire_gather(i6, b4):
            pltpu.async_copy(x_hbm.at[pk_v.at[i6, 0]], rows[b4], gsems[b4])

        def wait_gather(b4):
            pltpu.make_async_copy(x_hbm.at[pl.ds(0, _S)], rows[b4],
                                  gsems[b4]).wait()

        def fire_scatter(b4, b6):
            pltpu.async_copy(rows[b4], agg_sh.at[pk_v.at[b6, 1]], ssems[b4],
                             add=True)

        def wait_scatter(b4):
            pltpu.make_async_copy(rows[b4], agg_sh.at[pl.ds(0, _S)],
                                  ssems[b4]).wait()

        def scale(b4, w6):
            def grp(g, carry):
                w16 = w_v[w6, pl.ds(g * _L, _L)]
                for e in range(_L):
                    ws = _splat(w16, e)
                    row = g * _L + e
                    for j in range(_D // _L):
                        rows[b4][row, pl.ds(j * _L, _L)] = (
                            rows[b4][row, pl.ds(j * _L, _L)] * ws)
                return carry
            lax.fori_loop(0, _S // _L, grp, 0)

        fire_idx(0, 0)
        fire_idx(1, 1)
        fire_idx(2, 2)
        wait_idx(0)
        fire_gather(0, 0)
        wait_idx(1)
        fire_gather(1, 1)

        zc.wait()

        @pl.when(s == _NS - 1)
        def _zero_tail():
            pltpu.async_copy(z_hbm.at[pl.ds(_NS * _RPT, _TAIL)],
                             agg_sh.at[pl.ds(_NS * _RPT, _TAIL)], zsem).wait()

        plsc.subcore_barrier()

        def body(i, carry):
            for h in range(6):
                kk = 6 * i + h
                b3 = h % 3
                g3 = (h + 2) % 3
                b6 = h % 6
                n6 = (h + 2) % 6

                if h < 1:
                    @pl.when(kk >= 1)
                    def _ws():
                        wait_scatter(g3)
                else:
                    wait_scatter(g3)
                wait_idx(n6)
                fire_gather(n6, g3)
                wait_gather(b3)
                scale(b3, b6)
                fire_scatter(b3, b6)
                fire_idx(kk + 3, (h + 3) % 6)
            return carry

        lax.fori_loop(0, _NCH // 6, body, 0)

        wait_scatter((_NCH - 1) % 3)
        wait_gather(_NCH % 3)
        wait_gather((_NCH + 1) % 3)
        wait_idx((_NCH + 2) % 6)

        plsc.subcore_barrier()
        pltpu.sync_copy(agg_sh.at[pl.ds(s * _RPT, _RPT)],
                        out_hbm.at[c, pl.ds(s * _RPT, _RPT)])

        @pl.when(s == _NS - 1)
        def _out_tail():
            pltpu.sync_copy(agg_sh.at[pl.ds(_NS * _RPT, _TAIL)],
                            out_hbm.at[c, pl.ds(_NS * _RPT, _TAIL)])

    return k(x, srcf, dstf, wf, zeros)


def _tc_epilogue(p0, p1, x0, xin, w1):
    blk = 1000

    def body(p0_ref, p1_ref, x0_ref, xin_ref, w1_ref, o_ref):
        agg = p0_ref[...] + p1_ref[...]
        h = (1.0 - _ALPHA) * agg + _ALPHA * x0_ref[...]
        hw = jnp.dot(h, w1_ref[...], preferred_element_type=jnp.float32)
        o_ref[...] = jnp.maximum(
            (1.0 - _BETA) * h + _BETA * hw + xin_ref[...], 0.0)

    return pl.pallas_call(
        body,
        grid=(_N // blk,),
        in_specs=[
            pl.BlockSpec((blk, _D), lambda i: (i, 0)),
            pl.BlockSpec((blk, _D), lambda i: (i, 0)),
            pl.BlockSpec((blk, _D), lambda i: (i, 0)),
            pl.BlockSpec((blk, _D), lambda i: (i, 0)),
            pl.BlockSpec((_D, _D), lambda i: (0, 0)),
        ],
        out_specs=pl.BlockSpec((blk, _D), lambda i: (i, 0)),
        out_shape=jax.ShapeDtypeStruct((_N, _D), jnp.float32),
    )(p0, p1, x0, xin, w1)


def kernel(x, x_0, edge_index, edge_weight, W1):
    src = edge_index[0].astype(jnp.int32)
    dst = edge_index[1].astype(jnp.int32)

    pad = _E2 - _E + 3 * _S
    fill = (jnp.arange(pad, dtype=jnp.int32) * 13) % _N
    srcf = jnp.concatenate([src, fill])
    dstf = jnp.concatenate([dst, fill])
    wf = jnp.concatenate([edge_weight, jnp.zeros((pad,), jnp.float32)])

    zeros = jnp.zeros((_N, _D), jnp.float32)
    partials = _sc_gather_scatter(x, srcf, dstf, wf, zeros)
    return _tc_epilogue(partials[0], partials[1], x_0, x, W1)
